# baseline (speedup 1.0000x reference)
import functools

import jax
import jax.numpy as jnp
from jax.experimental import pallas as pl
from jax.experimental.pallas import tpu as pltpu


_N_PROJ = 8192
_N_PC = 8192
_D = 128

_BI = 512
_BJ = 2048
_NI = _N_PROJ // _BI
_NJ = _N_PC // _BJ


def _dist_loss_kernel(proj_ref, pct_ref, pe_ref, mp_ref, ft_ref, mp_out_ref,
                      minacc_ref):
    j = pl.program_id(1)

    a = proj_ref[...]
    bt = pct_ref[...]
    ab = jax.lax.dot_general(
        a.astype(jnp.bfloat16), bt.astype(jnp.bfloat16),
        dimension_numbers=(((1,), (0,)), ((), ())),
        preferred_element_type=jnp.float32)
    bb = jnp.sum(bt * bt, axis=0, keepdims=True)
    part = bb - 2.0 * ab
    pm = jnp.min(part, axis=1, keepdims=True)

    @pl.when(j == 0)
    def _():
        minacc_ref[...] = pm

    @pl.when(j > 0)
    def _():
        minacc_ref[...] = jnp.minimum(minacc_ref[...], pm)

    @pl.when(j == _NJ - 1)
    def _():
        aa = jnp.sum(a * a, axis=1, keepdims=True)
        d = minacc_ref[...] + aa
        ft = jnp.abs(jnp.sqrt(jnp.abs(d) + 1e-7) - jnp.abs(pe_ref[...]))
        ft_ref[...] = ft
        mp_out_ref[...] = jnp.abs(mp_ref[...])


@functools.partial(jax.jit, static_argnames=("interpret",))
def _dist_loss(proj, pc_t, proj_eval, manifold, interpret=False):
    ft, mp_abs = pl.pallas_call(
        _dist_loss_kernel,
        grid=(_NI, _NJ),
        in_specs=[
            pl.BlockSpec((_BI, _D), lambda i, j: (i, 0)),
            pl.BlockSpec((_D, _BJ), lambda i, j: (0, j)),
            pl.BlockSpec((_BI, 1), lambda i, j: (i, 0)),
            pl.BlockSpec((_BI, 1), lambda i, j: (i, 0)),
        ],
        out_specs=[
            pl.BlockSpec((_BI, 1), lambda i, j: (i, 0)),
            pl.BlockSpec((_BI, 1), lambda i, j: (i, 0)),
        ],
        out_shape=[
            jax.ShapeDtypeStruct((_N_PROJ, 1), jnp.float32),
            jax.ShapeDtypeStruct((_N_PROJ, 1), jnp.float32),
        ],
        scratch_shapes=[pltpu.VMEM((_BI, 1), jnp.float32)],
        compiler_params=pltpu.CompilerParams(
            dimension_semantics=("parallel", "arbitrary"),
        ),
        interpret=interpret,
    )(proj, pc_t, proj_eval, manifold)
    return ft, mp_abs


def kernel(zerolevelset_points, genlevelset_points, pc_input,
           zerolevelset_eval, gen_points_eval, manifold_pnts_pred,
           loss_lambda):
    if zerolevelset_points is not None:
        proj = jnp.concatenate([zerolevelset_points, genlevelset_points], axis=0)
        proj_eval = jnp.concatenate([zerolevelset_eval, gen_points_eval], axis=0)
    else:
        proj = genlevelset_points
        proj_eval = gen_points_eval
    ft, mp_abs = _dist_loss(proj, pc_input.T, proj_eval, manifold_pnts_pred)
    mean_first = jnp.mean(ft)
    mean_second = jnp.mean(mp_abs)
    ll = 0.1 if loss_lambda is None else loss_lambda
    loss = mean_first + ll * mean_second
    return (loss, mean_first, mean_second)
